# baseline (speedup 1.0000x reference)
import dataclasses
import functools

import jax
import jax.numpy as jnp
from jax import lax
from jax.experimental import pallas as pl
from jax.experimental.pallas import tpu as pltpu
from jax.experimental.pallas import tpu_sc as plsc

N = 10000
N_PAD = 10240
E = 320000
IN_F = 128
HID = 128
CLS = 64

NC = 2
NS = 16
NW = NC * NS
K = 128
NCHUNK = 80
E_PER_W = K * NCHUNK
E_PAD = E_PER_W * NW
RPS = N_PAD // NS

BLK = 1024


HR = N_PAD // 128


def _sc_degree_v(dst):
    mesh = plsc.VectorSubcoreMesh(core_axis_name="c", subcore_axis_name="s")
    zrows = jnp.zeros((HR, 128), jnp.float32)
    rowid = jnp.arange(HR, dtype=jnp.int32)
    cp = pltpu.CompilerParams()
    if "needs_layout_passes" in pltpu.CompilerParams.__dataclass_fields__:
        cp = dataclasses.replace(cp, needs_layout_passes=False)

    @functools.partial(
        pl.kernel,
        out_type=jax.ShapeDtypeStruct((NC, HR, 128), jnp.float32),
        mesh=mesh,
        compiler_params=cp,
        scratch_types=[
            pltpu.VMEM((E_PER_W,), jnp.int32),
            pltpu.VMEM((HR, 128), jnp.float32),
            pltpu.VMEM((HR,), jnp.int32),
            pltpu.VMEM_SHARED((HR, 128), jnp.float32),
        ],
    )
    def k(dst_hbm, z_hbm, rowid_hbm, out_hbm, dst_v, hist_v, rid_v, acc):
        c = lax.axis_index("c")
        s = lax.axis_index("s")
        w = c * NS + s
        pltpu.sync_copy(z_hbm, hist_v)
        pltpu.sync_copy(rowid_hbm, rid_v)
        pltpu.sync_copy(dst_hbm.at[pl.ds(w * E_PER_W, E_PER_W)], dst_v)

        @pl.when(s == 0)
        def _():
            pltpu.sync_copy(z_hbm, acc)

        ones = jnp.ones((16,), jnp.float32)

        @pl.loop(0, E_PER_W, step=64)
        def _(i):
            for u in range(4):
                idx = dst_v[pl.ds(i + u * 16, 16)]
                r = lax.shift_right_logical(idx, 7)
                col = lax.bitwise_and(idx, 127)
                plsc.addupdate_scatter(hist_v, [r, col], ones)

        plsc.subcore_barrier()
        pltpu.sync_copy(hist_v, acc.at[rid_v], add=True)
        plsc.subcore_barrier()

        @pl.when(s == 0)
        def _():
            pltpu.sync_copy(acc, out_hbm.at[c])

    return k(dst, zrows, rowid)


def _sc_scatter(table, src, dst, d, ks, nbuf):
    mesh = plsc.VectorSubcoreMesh(core_axis_name="c", subcore_axis_name="s")
    zrows = jnp.zeros((RPS, d), jnp.float32)
    nchunk = E_PER_W // ks
    src2d = src.reshape(NW, nchunk, ks)

    @functools.partial(
        pl.kernel,
        out_type=jax.ShapeDtypeStruct((NC, N_PAD, d), jnp.float32),
        mesh=mesh,
        scratch_types=[
            pltpu.VMEM((nchunk, ks), jnp.int32),
            [pltpu.VMEM((ks,), jnp.int32) for _ in range(nbuf)],
            [pltpu.VMEM((ks, d), jnp.float32) for _ in range(nbuf)],
            pltpu.VMEM_SHARED((N_PAD, d), jnp.float32),
            [pltpu.SemaphoreType.DMA for _ in range(nbuf)],
            [pltpu.SemaphoreType.DMA for _ in range(nbuf)],
            [pltpu.SemaphoreType.DMA for _ in range(nbuf)],
        ],
    )
    def k(table_hbm, src_hbm, dst_hbm, z_hbm, out_hbm,
          src_all, dst_vs, rows_vs, acc, gsems, ssems, dsems):
        c = lax.axis_index("c")
        s = lax.axis_index("s")
        w = c * NS + s
        ebase = w * E_PER_W
        pltpu.sync_copy(src_hbm.at[w], src_all)
        pltpu.sync_copy(z_hbm, acc.at[pl.ds(s * RPS, RPS)])
        plsc.subcore_barrier()

        def start_gather(b, chunk):
            pltpu.make_async_copy(dst_hbm.at[pl.ds(ebase + chunk * ks, ks)],
                                  dst_vs[b], dsems[b]).start()
            pltpu.make_async_copy(table_hbm.at[src_all.at[chunk]], rows_vs[b],
                                  gsems[b]).start()

        def start_scatter(b, chunk):
            pltpu.make_async_copy(dst_hbm.at[pl.ds(ebase + chunk * ks, ks)],
                                  dst_vs[b], dsems[b]).wait()
            pltpu.make_async_copy(table_hbm.at[src_all.at[chunk]], rows_vs[b],
                                  gsems[b]).wait()
            pltpu.make_async_copy(rows_vs[b], acc.at[dst_vs[b]],
                                  ssems[b]).start(add=True)

        def wait_scatter(b):
            pltpu.make_async_copy(rows_vs[b], acc.at[dst_vs[b]],
                                  ssems[b]).wait()

        for b in range(nbuf):
            start_gather(b, b)

        @pl.loop(0, nchunk - nbuf, step=nbuf)
        def _(i):
            for b in range(nbuf):
                start_scatter(b, i + b)
            for b in range(nbuf):
                wait_scatter(b)
                start_gather(b, i + b + nbuf)

        for b in range(nbuf):
            start_scatter(b, nchunk - nbuf + b)
        for b in range(nbuf):
            wait_scatter(b)

        plsc.subcore_barrier()
        pltpu.sync_copy(acc.at[pl.ds(s * RPS, RPS)],
                        out_hbm.at[c, pl.ds(s * RPS, RPS)])

    return k(table, src2d, dst, zrows)


def _tc_matmul1(x_pad, w1):
    def body(x_ref, w_ref, h_ref):
        h_ref[...] = jnp.dot(x_ref[...], w_ref[...],
                             preferred_element_type=jnp.float32)

    return pl.pallas_call(
        body,
        grid=(N_PAD // BLK,),
        in_specs=[
            pl.BlockSpec((BLK, IN_F), lambda i: (i, 0)),
            pl.BlockSpec((IN_F, HID), lambda i: (0, 0)),
        ],
        out_specs=pl.BlockSpec((BLK, HID), lambda i: (i, 0)),
        out_shape=jax.ShapeDtypeStruct((N_PAD, HID), jnp.float32),
    )(x_pad, w1)


def _tc_scale(deg_p, h):
    def body(degp_ref, h_ref, g_ref, dinv_ref):
        deg = degp_ref[0] + degp_ref[1] + 1.0
        dinv = lax.rsqrt(deg)
        g_ref[...] = h_ref[...] * dinv[:, None]
        dinv_ref[...] = dinv

    return pl.pallas_call(
        body,
        grid=(N_PAD // BLK,),
        in_specs=[
            pl.BlockSpec((NC, BLK), lambda i: (0, i)),
            pl.BlockSpec((BLK, HID), lambda i: (i, 0)),
        ],
        out_specs=[
            pl.BlockSpec((BLK, HID), lambda i: (i, 0)),
            pl.BlockSpec((BLK,), lambda i: (i,)),
        ],
        out_shape=[
            jax.ShapeDtypeStruct((N_PAD, HID), jnp.float32),
            jax.ShapeDtypeStruct((N_PAD,), jnp.float32),
        ],
    )(deg_p, h)


def _tc_layer2(s1_p, g1, dinv, b1, w2):
    def body(sp_ref, g1_ref, dinv_ref, b1_ref, w_ref, g2_ref):
        dinv = dinv_ref[...]
        z = (sp_ref[0] + sp_ref[1] + g1_ref[...]) * dinv[:, None] + b1_ref[...]
        z = jnp.maximum(z, 0.0)
        h = jnp.dot(z, w_ref[...], preferred_element_type=jnp.float32)
        g2_ref[...] = h * dinv[:, None]

    return pl.pallas_call(
        body,
        grid=(N_PAD // BLK,),
        in_specs=[
            pl.BlockSpec((NC, BLK, HID), lambda i: (0, i, 0)),
            pl.BlockSpec((BLK, HID), lambda i: (i, 0)),
            pl.BlockSpec((BLK,), lambda i: (i,)),
            pl.BlockSpec((HID,), lambda i: (0,)),
            pl.BlockSpec((HID, HID), lambda i: (0, 0)),
        ],
        out_specs=pl.BlockSpec((BLK, HID), lambda i: (i, 0)),
        out_shape=jax.ShapeDtypeStruct((N_PAD, HID), jnp.float32),
    )(s1_p, g1, dinv, b1, w2)


def _tc_out(s2_p, g2, dinv, b2):
    def body(sp_ref, g2_ref, dinv_ref, b2_ref, o_ref):
        o_ref[...] = ((sp_ref[0] + sp_ref[1] + g2_ref[...])
                      * dinv_ref[...][:, None] + b2_ref[...])

    return pl.pallas_call(
        body,
        grid=(N_PAD // BLK,),
        in_specs=[
            pl.BlockSpec((NC, BLK, HID), lambda i: (0, i, 0)),
            pl.BlockSpec((BLK, HID), lambda i: (i, 0)),
            pl.BlockSpec((BLK,), lambda i: (i,)),
            pl.BlockSpec((HID,), lambda i: (0,)),
        ],
        out_specs=pl.BlockSpec((BLK, HID), lambda i: (i, 0)),
        out_shape=jax.ShapeDtypeStruct((N_PAD, HID), jnp.float32),
    )(s2_p, g2, dinv, b2)


def kernel(x, edge_index, W1, b1, W2, b2):
    ei = edge_index.astype(jnp.int32)
    pad_idx = N + (jnp.arange(E_PAD - E, dtype=jnp.int32) % (N_PAD - N))
    src = jnp.concatenate([ei[0], pad_idx])
    dst = jnp.concatenate([ei[1], pad_idx])
    x_pad = jnp.pad(x, ((0, N_PAD - N), (0, 0)))
    w2_pad = jnp.pad(W2, ((0, 0), (0, HID - CLS)))
    b2_pad = jnp.pad(b2, ((0, HID - CLS),))

    deg_p = _sc_degree_v(dst).reshape(NC, N_PAD)
    h1 = _tc_matmul1(x_pad, W1)
    g1, dinv = _tc_scale(deg_p, h1)
    s1_p = _sc_scatter(g1, src, dst, HID, K, 2)
    g2 = _tc_layer2(s1_p, g1, dinv, b1, w2_pad)
    s2_p = _sc_scatter(g2, src, dst, HID, K, 2)
    out = _tc_out(s2_p, g2, dinv, b2_pad)
    return out[:N, :CLS]


# speedup vs baseline: 1.1438x; 1.1438x over previous
import dataclasses
import functools

import jax
import jax.numpy as jnp
from jax import lax
from jax.experimental import pallas as pl
from jax.experimental.pallas import tpu as pltpu
from jax.experimental.pallas import tpu_sc as plsc

N = 10000
N_PAD = 10240
E = 320000
IN_F = 128
HID = 128
CLS = 64

NC = 2
NS = 16
NW = NC * NS
K = 112
NCHUNK = 90
E_PER_W = K * NCHUNK
E_PAD = E_PER_W * NW
RPS = N_PAD // NS

BLK = 1024


HR = N_PAD // 128


def _sc_degree_v(dst):
    mesh = plsc.VectorSubcoreMesh(core_axis_name="c", subcore_axis_name="s")
    zrows = jnp.zeros((HR, 128), jnp.float32)
    rowid = jnp.arange(HR, dtype=jnp.int32)
    cp = pltpu.CompilerParams()
    if "needs_layout_passes" in pltpu.CompilerParams.__dataclass_fields__:
        cp = dataclasses.replace(cp, needs_layout_passes=False)

    @functools.partial(
        pl.kernel,
        out_type=jax.ShapeDtypeStruct((NC, HR, 128), jnp.float32),
        mesh=mesh,
        compiler_params=cp,
        scratch_types=[
            pltpu.VMEM((E_PER_W,), jnp.int32),
            pltpu.VMEM((HR, 128), jnp.float32),
            pltpu.VMEM((HR,), jnp.int32),
            pltpu.VMEM_SHARED((HR, 128), jnp.float32),
        ],
    )
    def k(dst_hbm, z_hbm, rowid_hbm, out_hbm, dst_v, hist_v, rid_v, acc):
        c = lax.axis_index("c")
        s = lax.axis_index("s")
        w = c * NS + s
        pltpu.sync_copy(z_hbm, hist_v)
        pltpu.sync_copy(rowid_hbm, rid_v)
        pltpu.sync_copy(dst_hbm.at[pl.ds(w * E_PER_W, E_PER_W)], dst_v)

        @pl.when(s == 0)
        def _():
            pltpu.sync_copy(z_hbm, acc)

        ones = jnp.ones((16,), jnp.float32)

        @pl.loop(0, E_PER_W, step=48)
        def _(i):
            for u in range(3):
                idx = dst_v[pl.ds(i + u * 16, 16)]
                r = lax.shift_right_logical(idx, 7)
                col = lax.bitwise_and(idx, 127)
                plsc.addupdate_scatter(hist_v, [r, col], ones)

        plsc.subcore_barrier()
        pltpu.sync_copy(hist_v, acc.at[rid_v], add=True)
        plsc.subcore_barrier()

        @pl.when(s == 0)
        def _():
            pltpu.sync_copy(acc, out_hbm.at[c])

    return k(dst, zrows, rowid)


def _sc_scatter(table, src, dst, d, ks, nbuf):
    mesh = plsc.VectorSubcoreMesh(core_axis_name="c", subcore_axis_name="s")
    zrows = jnp.zeros((RPS, d), jnp.float32)
    nchunk = E_PER_W // ks

    @functools.partial(
        pl.kernel,
        out_type=jax.ShapeDtypeStruct((NC, N_PAD, d), jnp.float32),
        mesh=mesh,
        scratch_types=[
            [pltpu.VMEM((ks,), jnp.int32) for _ in range(nbuf)],
            [pltpu.VMEM((ks,), jnp.int32) for _ in range(nbuf)],
            [pltpu.VMEM((ks, d), jnp.float32) for _ in range(nbuf)],
            pltpu.VMEM_SHARED((N_PAD, d), jnp.float32),
            [pltpu.SemaphoreType.DMA for _ in range(nbuf)],
            [pltpu.SemaphoreType.DMA for _ in range(nbuf)],
        ],
    )
    def k(table_hbm, src_hbm, dst_hbm, z_hbm, out_hbm,
          src_vs, dst_vs, rows_vs, acc, gsems, ssems):
        c = lax.axis_index("c")
        s = lax.axis_index("s")
        w = c * NS + s
        ebase = w * E_PER_W
        pltpu.sync_copy(z_hbm, acc.at[pl.ds(s * RPS, RPS)])
        plsc.subcore_barrier()

        def start_gather(b, chunk):
            pltpu.sync_copy(src_hbm.at[pl.ds(ebase + chunk * ks, ks)],
                            src_vs[b])
            pltpu.make_async_copy(table_hbm.at[src_vs[b]], rows_vs[b],
                                  gsems[b]).start()

        def start_scatter(b, chunk):
            pltpu.sync_copy(dst_hbm.at[pl.ds(ebase + chunk * ks, ks)],
                            dst_vs[b])
            pltpu.make_async_copy(table_hbm.at[src_vs[b]], rows_vs[b],
                                  gsems[b]).wait()
            pltpu.make_async_copy(rows_vs[b], acc.at[dst_vs[b]],
                                  ssems[b]).start(add=True)

        def wait_scatter(b):
            pltpu.make_async_copy(rows_vs[b], acc.at[dst_vs[b]],
                                  ssems[b]).wait()

        for b in range(nbuf):
            start_gather(b, b)

        @pl.loop(0, nchunk - nbuf, step=nbuf)
        def _(i):
            for b in range(nbuf):
                start_scatter(b, i + b)
            for b in range(nbuf):
                wait_scatter(b)
                start_gather(b, i + b + nbuf)

        for b in range(nbuf):
            start_scatter(b, nchunk - nbuf + b)
        for b in range(nbuf):
            wait_scatter(b)

        plsc.subcore_barrier()
        pltpu.sync_copy(acc.at[pl.ds(s * RPS, RPS)],
                        out_hbm.at[c, pl.ds(s * RPS, RPS)])

    return k(table, src, dst, zrows)


def _tc_matmul1(x_pad, w1):
    def body(x_ref, w_ref, h_ref):
        h_ref[...] = jnp.dot(x_ref[...], w_ref[...],
                             preferred_element_type=jnp.float32)

    return pl.pallas_call(
        body,
        grid=(N_PAD // BLK,),
        in_specs=[
            pl.BlockSpec((BLK, IN_F), lambda i: (i, 0)),
            pl.BlockSpec((IN_F, HID), lambda i: (0, 0)),
        ],
        out_specs=pl.BlockSpec((BLK, HID), lambda i: (i, 0)),
        out_shape=jax.ShapeDtypeStruct((N_PAD, HID), jnp.float32),
    )(x_pad, w1)


def _tc_scale(deg_p, h):
    def body(degp_ref, h_ref, g_ref, dinv_ref):
        deg = degp_ref[0] + degp_ref[1] + 1.0
        dinv = lax.rsqrt(deg)
        g_ref[...] = h_ref[...] * dinv[:, None]
        dinv_ref[...] = dinv

    return pl.pallas_call(
        body,
        grid=(N_PAD // BLK,),
        in_specs=[
            pl.BlockSpec((NC, BLK), lambda i: (0, i)),
            pl.BlockSpec((BLK, HID), lambda i: (i, 0)),
        ],
        out_specs=[
            pl.BlockSpec((BLK, HID), lambda i: (i, 0)),
            pl.BlockSpec((BLK,), lambda i: (i,)),
        ],
        out_shape=[
            jax.ShapeDtypeStruct((N_PAD, HID), jnp.float32),
            jax.ShapeDtypeStruct((N_PAD,), jnp.float32),
        ],
    )(deg_p, h)


def _tc_layer2(s1_p, g1, dinv, b1, w2):
    def body(sp_ref, g1_ref, dinv_ref, b1_ref, w_ref, g2_ref):
        dinv = dinv_ref[...]
        z = (sp_ref[0] + sp_ref[1] + g1_ref[...]) * dinv[:, None] + b1_ref[...]
        z = jnp.maximum(z, 0.0)
        h = jnp.dot(z, w_ref[...], preferred_element_type=jnp.float32)
        g2_ref[...] = h * dinv[:, None]

    return pl.pallas_call(
        body,
        grid=(N_PAD // BLK,),
        in_specs=[
            pl.BlockSpec((NC, BLK, HID), lambda i: (0, i, 0)),
            pl.BlockSpec((BLK, HID), lambda i: (i, 0)),
            pl.BlockSpec((BLK,), lambda i: (i,)),
            pl.BlockSpec((HID,), lambda i: (0,)),
            pl.BlockSpec((HID, HID), lambda i: (0, 0)),
        ],
        out_specs=pl.BlockSpec((BLK, HID), lambda i: (i, 0)),
        out_shape=jax.ShapeDtypeStruct((N_PAD, HID), jnp.float32),
    )(s1_p, g1, dinv, b1, w2)


def _tc_out(s2_p, g2, dinv, b2):
    def body(sp_ref, g2_ref, dinv_ref, b2_ref, o_ref):
        o_ref[...] = ((sp_ref[0] + sp_ref[1] + g2_ref[...])
                      * dinv_ref[...][:, None] + b2_ref[...])

    return pl.pallas_call(
        body,
        grid=(N_PAD // BLK,),
        in_specs=[
            pl.BlockSpec((NC, BLK, HID), lambda i: (0, i, 0)),
            pl.BlockSpec((BLK, HID), lambda i: (i, 0)),
            pl.BlockSpec((BLK,), lambda i: (i,)),
            pl.BlockSpec((HID,), lambda i: (0,)),
        ],
        out_specs=pl.BlockSpec((BLK, HID), lambda i: (i, 0)),
        out_shape=jax.ShapeDtypeStruct((N_PAD, HID), jnp.float32),
    )(s2_p, g2, dinv, b2)


def kernel(x, edge_index, W1, b1, W2, b2):
    ei = edge_index.astype(jnp.int32)
    pad_idx = N + (jnp.arange(E_PAD - E, dtype=jnp.int32) % (N_PAD - N))
    src = jnp.concatenate([ei[0], pad_idx])
    dst = jnp.concatenate([ei[1], pad_idx])
    x_pad = jnp.pad(x, ((0, N_PAD - N), (0, 0)))
    w2_pad = jnp.pad(W2, ((0, 0), (0, HID - CLS)))
    b2_pad = jnp.pad(b2, ((0, HID - CLS),))

    deg_p = _sc_degree_v(dst).reshape(NC, N_PAD)
    h1 = _tc_matmul1(x_pad, W1)
    g1, dinv = _tc_scale(deg_p, h1)
    s1_p = _sc_scatter(g1, src, dst, HID, K, 3)
    g2 = _tc_layer2(s1_p, g1, dinv, b1, w2_pad)
    s2_p = _sc_scatter(g2, src, dst, HID, K, 3)
    out = _tc_out(s2_p, g2, dinv, b2_pad)
    return out[:N, :CLS]


# speedup vs baseline: 1.1848x; 1.0358x over previous
import dataclasses
import functools

import jax
import jax.numpy as jnp
from jax import lax
from jax.experimental import pallas as pl
from jax.experimental.pallas import tpu as pltpu
from jax.experimental.pallas import tpu_sc as plsc

N = 10000
N_PAD = 10240
E = 320000
IN_F = 128
HID = 128
CLS = 64

NC = 2
NS = 16
NW = NC * NS
K = 112
NCHUNK = 90
E_PER_W = K * NCHUNK
E_PAD = E_PER_W * NW
RPS = N_PAD // NS

BLK = 1024


HR = N_PAD // 128


def _sc_degree_v(dst):
    mesh = plsc.VectorSubcoreMesh(core_axis_name="c", subcore_axis_name="s")
    zrows = jnp.zeros((HR, 128), jnp.float32)
    rowid = jnp.arange(HR, dtype=jnp.int32)
    cp = pltpu.CompilerParams()
    if "needs_layout_passes" in pltpu.CompilerParams.__dataclass_fields__:
        cp = dataclasses.replace(cp, needs_layout_passes=False)

    @functools.partial(
        pl.kernel,
        out_type=jax.ShapeDtypeStruct((NC, HR, 128), jnp.float32),
        mesh=mesh,
        compiler_params=cp,
        scratch_types=[
            pltpu.VMEM((E_PER_W,), jnp.int32),
            pltpu.VMEM((HR, 128), jnp.float32),
            pltpu.VMEM((HR,), jnp.int32),
            pltpu.VMEM_SHARED((HR, 128), jnp.float32),
        ],
    )
    def k(dst_hbm, z_hbm, rowid_hbm, out_hbm, dst_v, hist_v, rid_v, acc):
        c = lax.axis_index("c")
        s = lax.axis_index("s")
        w = c * NS + s
        pltpu.sync_copy(z_hbm, hist_v)
        pltpu.sync_copy(rowid_hbm, rid_v)
        pltpu.sync_copy(dst_hbm.at[pl.ds(w * E_PER_W, E_PER_W)], dst_v)

        @pl.when(s == 0)
        def _():
            pltpu.sync_copy(z_hbm, acc)

        ones = jnp.ones((16,), jnp.float32)

        @pl.loop(0, E_PER_W, step=48)
        def _(i):
            for u in range(3):
                idx = dst_v[pl.ds(i + u * 16, 16)]
                r = lax.shift_right_logical(idx, 7)
                col = lax.bitwise_and(idx, 127)
                plsc.addupdate_scatter(hist_v, [r, col], ones)

        plsc.subcore_barrier()
        pltpu.sync_copy(hist_v, acc.at[rid_v], add=True)
        plsc.subcore_barrier()

        @pl.when(s == 0)
        def _():
            pltpu.sync_copy(acc, out_hbm.at[c])

    return k(dst, zrows, rowid)


def _sc_scatter(table, src, dst, d, ks, nbuf):
    mesh = plsc.VectorSubcoreMesh(core_axis_name="c", subcore_axis_name="s")
    zrows = jnp.zeros((RPS, d), jnp.float32)
    nchunk = E_PER_W // ks

    @functools.partial(
        pl.kernel,
        out_type=jax.ShapeDtypeStruct((NC, N_PAD, d), jnp.float32),
        mesh=mesh,
        scratch_types=[
            [pltpu.VMEM((ks,), jnp.int32) for _ in range(nbuf)],
            [pltpu.VMEM((ks,), jnp.int32) for _ in range(nbuf)],
            [pltpu.VMEM((ks, d), jnp.float32) for _ in range(nbuf)],
            pltpu.VMEM_SHARED((N_PAD, d), jnp.float32),
            [pltpu.SemaphoreType.DMA for _ in range(nbuf)],
            [pltpu.SemaphoreType.DMA for _ in range(nbuf)],
            [pltpu.SemaphoreType.DMA for _ in range(nbuf)],
        ],
    )
    def k(table_hbm, src_hbm, dst_hbm, z_hbm, out_hbm,
          src_vs, dst_vs, rows_vs, acc, gsems, ssems, dsems):
        c = lax.axis_index("c")
        s = lax.axis_index("s")
        w = c * NS + s
        ebase = w * E_PER_W
        pltpu.sync_copy(z_hbm, acc.at[pl.ds(s * RPS, RPS)])
        plsc.subcore_barrier()

        def start_gather(b, chunk):
            pltpu.sync_copy(src_hbm.at[pl.ds(ebase + chunk * ks, ks)],
                            src_vs[b])
            pltpu.make_async_copy(table_hbm.at[src_vs[b]], rows_vs[b],
                                  gsems[b]).start()
            pltpu.make_async_copy(dst_hbm.at[pl.ds(ebase + chunk * ks, ks)],
                                  dst_vs[b], dsems[b]).start()

        def start_scatter(b, chunk):
            pltpu.make_async_copy(dst_hbm.at[pl.ds(ebase + chunk * ks, ks)],
                                  dst_vs[b], dsems[b]).wait()
            pltpu.make_async_copy(table_hbm.at[src_vs[b]], rows_vs[b],
                                  gsems[b]).wait()
            pltpu.make_async_copy(rows_vs[b], acc.at[dst_vs[b]],
                                  ssems[b]).start(add=True)

        def wait_scatter(b):
            pltpu.make_async_copy(rows_vs[b], acc.at[dst_vs[b]],
                                  ssems[b]).wait()

        for b in range(nbuf):
            start_gather(b, b)

        @pl.loop(0, nchunk - nbuf, step=nbuf)
        def _(i):
            for b in range(nbuf):
                start_scatter(b, i + b)
            for b in range(nbuf):
                wait_scatter(b)
                start_gather(b, i + b + nbuf)

        for b in range(nbuf):
            start_scatter(b, nchunk - nbuf + b)
        for b in range(nbuf):
            wait_scatter(b)

        plsc.subcore_barrier()
        pltpu.sync_copy(acc.at[pl.ds(s * RPS, RPS)],
                        out_hbm.at[c, pl.ds(s * RPS, RPS)])

    return k(table, src, dst, zrows)


def _tc_matmul1(x_pad, w1):
    def body(x_ref, w_ref, h_ref):
        h_ref[...] = jnp.dot(x_ref[...], w_ref[...],
                             preferred_element_type=jnp.float32)

    return pl.pallas_call(
        body,
        grid=(N_PAD // BLK,),
        in_specs=[
            pl.BlockSpec((BLK, IN_F), lambda i: (i, 0)),
            pl.BlockSpec((IN_F, HID), lambda i: (0, 0)),
        ],
        out_specs=pl.BlockSpec((BLK, HID), lambda i: (i, 0)),
        out_shape=jax.ShapeDtypeStruct((N_PAD, HID), jnp.float32),
    )(x_pad, w1)


def _tc_scale(deg_p, h):
    def body(degp_ref, h_ref, g_ref, dinv_ref):
        deg = degp_ref[0] + degp_ref[1] + 1.0
        dinv = lax.rsqrt(deg)
        g_ref[...] = h_ref[...] * dinv[:, None]
        dinv_ref[...] = dinv

    return pl.pallas_call(
        body,
        grid=(N_PAD // BLK,),
        in_specs=[
            pl.BlockSpec((NC, BLK), lambda i: (0, i)),
            pl.BlockSpec((BLK, HID), lambda i: (i, 0)),
        ],
        out_specs=[
            pl.BlockSpec((BLK, HID), lambda i: (i, 0)),
            pl.BlockSpec((BLK,), lambda i: (i,)),
        ],
        out_shape=[
            jax.ShapeDtypeStruct((N_PAD, HID), jnp.float32),
            jax.ShapeDtypeStruct((N_PAD,), jnp.float32),
        ],
    )(deg_p, h)


def _tc_layer2(s1_p, g1, dinv, b1, w2):
    def body(sp_ref, g1_ref, dinv_ref, b1_ref, w_ref, g2_ref):
        dinv = dinv_ref[...]
        z = (sp_ref[0] + sp_ref[1] + g1_ref[...]) * dinv[:, None] + b1_ref[...]
        z = jnp.maximum(z, 0.0)
        h = jnp.dot(z, w_ref[...], preferred_element_type=jnp.float32)
        g2_ref[...] = h * dinv[:, None]

    return pl.pallas_call(
        body,
        grid=(N_PAD // BLK,),
        in_specs=[
            pl.BlockSpec((NC, BLK, HID), lambda i: (0, i, 0)),
            pl.BlockSpec((BLK, HID), lambda i: (i, 0)),
            pl.BlockSpec((BLK,), lambda i: (i,)),
            pl.BlockSpec((HID,), lambda i: (0,)),
            pl.BlockSpec((HID, HID), lambda i: (0, 0)),
        ],
        out_specs=pl.BlockSpec((BLK, HID), lambda i: (i, 0)),
        out_shape=jax.ShapeDtypeStruct((N_PAD, HID), jnp.float32),
    )(s1_p, g1, dinv, b1, w2)


def _tc_out(s2_p, g2, dinv, b2):
    def body(sp_ref, g2_ref, dinv_ref, b2_ref, o_ref):
        o_ref[...] = ((sp_ref[0] + sp_ref[1] + g2_ref[...])
                      * dinv_ref[...][:, None] + b2_ref[...])

    return pl.pallas_call(
        body,
        grid=(N_PAD // BLK,),
        in_specs=[
            pl.BlockSpec((NC, BLK, HID), lambda i: (0, i, 0)),
            pl.BlockSpec((BLK, HID), lambda i: (i, 0)),
            pl.BlockSpec((BLK,), lambda i: (i,)),
            pl.BlockSpec((HID,), lambda i: (0,)),
        ],
        out_specs=pl.BlockSpec((BLK, HID), lambda i: (i, 0)),
        out_shape=jax.ShapeDtypeStruct((N_PAD, HID), jnp.float32),
    )(s2_p, g2, dinv, b2)


def kernel(x, edge_index, W1, b1, W2, b2):
    ei = edge_index.astype(jnp.int32)
    pad_idx = N + (jnp.arange(E_PAD - E, dtype=jnp.int32) % (N_PAD - N))
    src = jnp.concatenate([ei[0], pad_idx])
    dst = jnp.concatenate([ei[1], pad_idx])
    x_pad = jnp.pad(x, ((0, N_PAD - N), (0, 0)))
    w2_pad = jnp.pad(W2, ((0, 0), (0, HID - CLS)))
    b2_pad = jnp.pad(b2, ((0, HID - CLS),))

    deg_p = _sc_degree_v(dst).reshape(NC, N_PAD)
    h1 = _tc_matmul1(x_pad, W1)
    g1, dinv = _tc_scale(deg_p, h1)
    s1_p = _sc_scatter(g1, src, dst, HID, K, 3)
    g2 = _tc_layer2(s1_p, g1, dinv, b1, w2_pad)
    s2_p = _sc_scatter(g2, src, dst, HID, K, 3)
    out = _tc_out(s2_p, g2, dinv, b2_pad)
    return out[:N, :CLS]
